# Initial kernel scaffold; baseline (speedup 1.0000x reference)
import jax
import jax.numpy as jnp
from jax.experimental import pallas as pl


def kernel(query, key, value):
    raise NotImplementedError("write your pallas kernel here")


# speedup vs baseline: 3.6745x; 3.6745x over previous
import math
from functools import partial

import jax
import jax.numpy as jnp
from jax.experimental import pallas as pl
from jax.experimental.pallas import tpu as pltpu

_B, _S, _H, _E = 2, 4096, 16, 64
_ROUNDS = 5
_NEG_INF = float("-inf")


def _round_body(k_ref, v_ref, u_ref, shift_ref, bsqd_ref, flip_ref,
                ko_ref, vo_ref, kd_scr, f_scr, km_scr, kd1_scr, *, r, scale):
    b = 16 << r
    m = b // 2
    S = _S >> r
    A = S // b

    shift = shift_ref[0, 0, 0]
    bsqd = bsqd_ref[0, 0, 0]
    flipf = flip_ref[0, 0, 0]

    def rows(ref, idx):
        return ref[0, idx, :]

    def krows(idx):
        x = rows(k_ref, idx)
        if scale is not None:
            x = x * scale
        return x

    def gram(x):
        return jax.lax.dot_general(x, x, (((1,), (1,)), ((), ())))

    def km_of(gk, gv):
        return (gv + bsqd) * jnp.exp(gk - shift)

    def double_diff(km):
        n = km.shape[0]
        nb = n // 128
        for cb in range(nb):
            km_scr[cb] = km[:, cb * 128:(cb + 1) * 128]
        parts = [km_scr[cb, pl.Slice(0, n // 2, 2), :]
                 - km_scr[cb, pl.Slice(1, n // 2, 2), :] for cb in range(nb)]
        p = parts[0] if nb == 1 else jnp.concatenate(parts, axis=1)
        kd1t = jnp.transpose(p, (1, 0))
        if n // 2 == 128:
            kd1_scr[...] = kd1t
        else:
            kd1_scr[:, 0:n // 2] = kd1t
        full = (kd1_scr[pl.Slice(0, n // 2, 2), :]
                - kd1_scr[pl.Slice(1, n // 2, 2), :])
        return full if n // 2 == 128 else full[:, 0:n // 2]

    csd0_parts = []
    diag_parts = []
    if b <= 128:
        g = 128 // b
        n_chunks = S // 128
        eye = (jax.lax.broadcasted_iota(jnp.int32, (64, 64), 0)
               == jax.lax.broadcasted_iota(jnp.int32, (64, 64), 1)
               ).astype(jnp.float32)
        for c in range(n_chunks):
            kc = krows(pl.ds(c * 128, 128))
            vc = rows(v_ref, pl.ds(c * 128, 128))
            km = km_of(gram(kc), gram(vc))
            kd = double_diff(km)
            dg = jnp.sum(kd * eye, axis=0, keepdims=True)
            blocks = []
            for j in range(g):
                lo = j * m
                blocks.append(kd[lo:lo + m, lo:lo + m].reshape(1, m, m))
                csd0_parts.append(kd[lo:lo + 1, lo:lo + m])
                diag_parts.append(dg[:, lo:lo + m])
            kd_scr[pl.ds(c * g, g)] = jnp.concatenate(blocks, axis=0)
    else:
        kc = krows(pl.ds(0, 256))
        vc = rows(v_ref, pl.ds(0, 256))
        km = km_of(gram(kc), gram(vc))
        kd = double_diff(km)
        eye = (jax.lax.broadcasted_iota(jnp.int32, (m, m), 0)
               == jax.lax.broadcasted_iota(jnp.int32, (m, m), 1)
               ).astype(jnp.float32)
        kd_scr[0] = kd
        csd0_parts.append(kd[0:1, :])
        diag_parts.append(jnp.sum(kd * eye, axis=0, keepdims=True))

    csd0 = jnp.concatenate(csd0_parts, axis=0) if A > 1 else csd0_parts[0]
    diag = jnp.concatenate(diag_parts, axis=0) if A > 1 else diag_parts[0]
    rt = jnp.sqrt(diag)

    cmax = rt
    sh = 1
    while sh < m:
        shifted = jnp.concatenate(
            [jnp.full((A, sh), _NEG_INF, jnp.float32), cmax[:, :m - sh]], axis=1)
        cmax = jnp.maximum(cmax, shifted)
        sh *= 2
    u2 = u_ref[0] * rt * cmax

    lane = jax.lax.broadcasted_iota(jnp.int32, (A, m), 1)

    def body(i, carry):
        csd, swp = carry
        msk = (lane == i).astype(jnp.float32)
        csd_col = jnp.sum(csd * msk, axis=1, keepdims=True)
        u_col = jnp.sum(u2 * msk, axis=1, keepdims=True)
        spf = (u_col <= csd_col).astype(jnp.float32)
        swp = swp + jnp.abs(spf - flipf) * msk
        kd_i = kd_scr[:, pl.ds(i, 1), :].reshape(A, m)
        return csd + (1.0 - 2.0 * spf) * kd_i, swp

    swp0 = jnp.where(lane == 0, flipf, 0.0)
    _, swp = jax.lax.fori_loop(1, m, body, (csd0, swp0))
    f_scr[...] = swp

    for j in range(m):
        fj = f_scr[:, j:j + 1] > 0.5
        ke = krows(pl.Slice(2 * j, A, 2 * m))
        ko = krows(pl.Slice(2 * j + 1, A, 2 * m))
        ko_ref[0, pl.Slice(j, A, m), :] = jnp.where(fj, ko, ke)
        ve = rows(v_ref, pl.Slice(2 * j, A, 2 * m))
        vo = rows(v_ref, pl.Slice(2 * j + 1, A, 2 * m))
        vo_ref[0, pl.Slice(j, A, m), :] = jnp.where(fj, vo, ve)


def _attn_body(q_ref, kc_ref, vc_ref, o_ref, *, st):
    hh = pl.program_id(2)
    q = q_ref[0, :, hh, :] * st
    kc = kc_ref[0]
    vc = vc_ref[0]
    s = jax.lax.dot_general(q, kc, (((1,), (1,)), ((), ())))
    mx = jnp.max(s, axis=1, keepdims=True)
    p = jnp.exp(s - mx)
    p = p / jnp.sum(p, axis=1, keepdims=True)
    o_ref[0, :, hh, :] = jax.lax.dot_general(p, vc, (((1,), (0,)), ((), ())))


def _setup_randomness():
    rng = jax.random.key(42)
    halve_prob = 0.5 / 4096.0
    us, flips = [], []
    bucket, scur = 16, 4096
    for i in range(_ROUNDS):
        sb = bucket
        delta = halve_prob * sb * sb
        log_mult = 0.5 + math.log(2 * sb / delta)
        a_r, m_r = scur // bucket, bucket // 2
        r_i = jax.random.fold_in(rng, i)
        k1, k2 = jax.random.split(r_i)
        u = jax.random.uniform(k1, (_B, a_r, _H, m_r),
                               minval=-log_mult, maxval=log_mult)
        flip = jax.random.bernoulli(k2, 0.5, (1,))
        us.append(jnp.transpose(u, (0, 2, 1, 3)).reshape(_B * _H, a_r, m_r))
        flips.append(flip.astype(jnp.float32).reshape(1, 1, 1))
        bucket *= 2
        scur //= 2
    return us, flips


def kernel(query, key, value):
    B, T, H, E = query.shape
    assert (B, T, H, E) == (_B, _S, _H, _E)
    softmax_temp = 1.0 / math.sqrt(E)
    st = math.sqrt(softmax_temp)

    us, flips = _setup_randomness()

    k_scaled_sq = jnp.sum((key * st) * (key * st), axis=3, keepdims=True)
    shift = jnp.max(k_scaled_sq, axis=1, keepdims=True)
    shift = shift[:, 0, :, 0].reshape(B * H, 1, 1)
    bsqd = (jnp.max(jnp.abs(value), axis=(1, 3), keepdims=True)) ** 2
    bsqd = bsqd[:, 0, :, 0].reshape(B * H, 1, 1)

    bh = B * H
    k_cur = jnp.transpose(key, (0, 2, 1, 3)).reshape(bh, T, E)
    v_cur = jnp.transpose(value, (0, 2, 1, 3)).reshape(bh, T, E)
    for r in range(_ROUNDS):
        s_r = _S >> r
        b = 16 << r
        m = b // 2
        a_r = s_r // b
        in_spec = pl.BlockSpec((1, s_r, E), lambda i: (i, 0, 0))
        in_specs = [
                in_spec,
                in_spec,
                pl.BlockSpec((1, a_r, m), lambda i: (i, 0, 0)),
                pl.BlockSpec((1, 1, 1), lambda i: (i, 0, 0)),
                pl.BlockSpec((1, 1, 1), lambda i: (i, 0, 0)),
                pl.BlockSpec((1, 1, 1), lambda i: (0, 0, 0)),
        ]
        out_specs = [
            pl.BlockSpec((1, s_r // 2, E), lambda i: (i, 0, 0)),
            pl.BlockSpec((1, s_r // 2, E), lambda i: (i, 0, 0)),
        ]
        k_cur, v_cur = pl.pallas_call(
            partial(_round_body, r=r, scale=st if r == 0 else None),
            grid=(bh,),
            in_specs=in_specs,
            out_specs=out_specs,
            out_shape=[
                jax.ShapeDtypeStruct((bh, s_r // 2, E), jnp.float32),
                jax.ShapeDtypeStruct((bh, s_r // 2, E), jnp.float32),
            ],
            scratch_shapes=[
                pltpu.VMEM((a_r, m, m), jnp.float32),
                pltpu.VMEM((a_r, m), jnp.float32),
                pltpu.VMEM((2, 256, 128) if b > 128 else (1, 128, 128),
                           jnp.float32),
                pltpu.VMEM((256, 128) if b > 128 else (128, 128), jnp.float32),
            ],
            compiler_params=pltpu.CompilerParams(
                dimension_semantics=("arbitrary",),
            ),
        )(k_cur, v_cur, us[r], shift, bsqd, flips[r])

    tq = 512
    out = pl.pallas_call(
        partial(_attn_body, st=st),
        grid=(B, T // tq, H),
        in_specs=[
            pl.BlockSpec((1, tq, H, E), lambda b_, t, h: (b_, t, 0, 0)),
            pl.BlockSpec((1, 128, E), lambda b_, t, h: (b_ * _H + h, 0, 0)),
            pl.BlockSpec((1, 128, E), lambda b_, t, h: (b_ * _H + h, 0, 0)),
        ],
        out_specs=pl.BlockSpec((1, tq, H, E), lambda b_, t, h: (b_, t, 0, 0)),
        out_shape=jax.ShapeDtypeStruct((B, T, H, E), jnp.float32),
        compiler_params=pltpu.CompilerParams(
            dimension_semantics=("arbitrary", "arbitrary", "arbitrary"),
        ),
    )(query, k_cur, v_cur)
    return out


# speedup vs baseline: 3.9917x; 1.0863x over previous
import math
from functools import partial

import jax
import jax.numpy as jnp
from jax.experimental import pallas as pl
from jax.experimental.pallas import tpu as pltpu

_B, _S, _H, _E = 2, 4096, 16, 64
_ROUNDS = 5
_NEG_INF = float("-inf")


def _round_body(k_ref, v_ref, u_ref, shift_ref, bsqd_ref, flip_ref,
                ko_ref, vo_ref, kd_scr, f_scr, km_scr, kd1_scr, *, r, scale):
    b = 16 << r
    m = b // 2
    S = _S >> r
    A = S // b

    shift = shift_ref[0, 0, 0]
    bsqd = bsqd_ref[0, 0, 0]
    flipf = flip_ref[0, 0, 0]

    def rows(ref, idx):
        return ref[0, idx, :]

    def krows(idx):
        x = rows(k_ref, idx)
        if scale is not None:
            x = x * scale
        return x

    def gram(x):
        return jax.lax.dot_general(x, x, (((1,), (1,)), ((), ())))

    def km_of(gk, gv):
        return (gv + bsqd) * jnp.exp(gk - shift)

    def double_diff(km):
        n = km.shape[0]
        nb = n // 128
        for cb in range(nb):
            km_scr[cb] = km[:, cb * 128:(cb + 1) * 128]
        parts = [km_scr[cb, pl.Slice(0, n // 2, 2), :]
                 - km_scr[cb, pl.Slice(1, n // 2, 2), :] for cb in range(nb)]
        p = parts[0] if nb == 1 else jnp.concatenate(parts, axis=1)
        kd1t = jnp.transpose(p, (1, 0))
        if n // 2 == 128:
            kd1_scr[...] = kd1t
        else:
            kd1_scr[:, 0:n // 2] = kd1t
        full = (kd1_scr[pl.Slice(0, n // 2, 2), :]
                - kd1_scr[pl.Slice(1, n // 2, 2), :])
        return full if n // 2 == 128 else full[:, 0:n // 2]

    csd0_parts = []
    diag_parts = []
    if b <= 128:
        g = 128 // b
        n_chunks = S // 128
        eye = (jax.lax.broadcasted_iota(jnp.int32, (64, 64), 0)
               == jax.lax.broadcasted_iota(jnp.int32, (64, 64), 1)
               ).astype(jnp.float32)
        for c in range(n_chunks):
            kc = krows(pl.ds(c * 128, 128))
            vc = rows(v_ref, pl.ds(c * 128, 128))
            km = km_of(gram(kc), gram(vc))
            kd = double_diff(km)
            dg = jnp.sum(kd * eye, axis=0, keepdims=True)
            blocks = []
            for j in range(g):
                lo = j * m
                blocks.append(kd[lo:lo + m, lo:lo + m].reshape(1, m, m))
                csd0_parts.append(kd[lo:lo + 1, lo:lo + m])
                diag_parts.append(dg[:, lo:lo + m])
            kd_scr[pl.ds(c * g, g)] = jnp.concatenate(blocks, axis=0)
    else:
        kc = krows(pl.ds(0, 256))
        vc = rows(v_ref, pl.ds(0, 256))
        km = km_of(gram(kc), gram(vc))
        kd = double_diff(km)
        eye = (jax.lax.broadcasted_iota(jnp.int32, (m, m), 0)
               == jax.lax.broadcasted_iota(jnp.int32, (m, m), 1)
               ).astype(jnp.float32)
        kd_scr[0] = kd
        csd0_parts.append(kd[0:1, :])
        diag_parts.append(jnp.sum(kd * eye, axis=0, keepdims=True))

    csd0 = jnp.concatenate(csd0_parts, axis=0) if A > 1 else csd0_parts[0]
    diag = jnp.concatenate(diag_parts, axis=0) if A > 1 else diag_parts[0]
    rt = jnp.sqrt(diag)

    cmax = rt
    sh = 1
    while sh < m:
        shifted = jnp.concatenate(
            [jnp.full((A, sh), _NEG_INF, jnp.float32), cmax[:, :m - sh]], axis=1)
        cmax = jnp.maximum(cmax, shifted)
        sh *= 2
    u2 = u_ref[0] * rt * cmax

    lane = jax.lax.broadcasted_iota(jnp.int32, (A, m), 1)

    def body(i, carry):
        csd, swp = carry
        msk = (lane == i).astype(jnp.float32)
        csd_col = jnp.sum(csd * msk, axis=1, keepdims=True)
        u_col = jnp.sum(u2 * msk, axis=1, keepdims=True)
        spf = (u_col <= csd_col).astype(jnp.float32)
        swp = swp + jnp.abs(spf - flipf) * msk
        kd_i = kd_scr[:, pl.ds(i, 1), :].reshape(A, m)
        return csd + (1.0 - 2.0 * spf) * kd_i, swp

    swp0 = jnp.where(lane == 0, flipf, 0.0)
    _, swp = jax.lax.fori_loop(1, m, body, (csd0, swp0))
    f_scr[...] = swp

    for j in range(m):
        fj = f_scr[:, j:j + 1] > 0.5
        ke = krows(pl.Slice(2 * j, A, 2 * m))
        ko = krows(pl.Slice(2 * j + 1, A, 2 * m))
        ko_ref[0, pl.Slice(j, A, m), :] = jnp.where(fj, ko, ke)
        ve = rows(v_ref, pl.Slice(2 * j, A, 2 * m))
        vo = rows(v_ref, pl.Slice(2 * j + 1, A, 2 * m))
        vo_ref[0, pl.Slice(j, A, m), :] = jnp.where(fj, vo, ve)


def _attn_body(q_ref, kc_ref, vc_ref, o_ref, *, st):
    for h in range(_H):
        q = q_ref[0, :, h, :] * st
        kc = kc_ref[h]
        vc = vc_ref[h]
        s = jax.lax.dot_general(q, kc, (((1,), (1,)), ((), ())))
        mx = jnp.max(s, axis=1, keepdims=True)
        p = jnp.exp(s - mx)
        p = p / jnp.sum(p, axis=1, keepdims=True)
        o_ref[0, :, h, :] = jax.lax.dot_general(p, vc, (((1,), (0,)), ((), ())))


def _setup_randomness():
    rng = jax.random.key(42)
    halve_prob = 0.5 / 4096.0
    us, flips = [], []
    bucket, scur = 16, 4096
    for i in range(_ROUNDS):
        sb = bucket
        delta = halve_prob * sb * sb
        log_mult = 0.5 + math.log(2 * sb / delta)
        a_r, m_r = scur // bucket, bucket // 2
        r_i = jax.random.fold_in(rng, i)
        k1, k2 = jax.random.split(r_i)
        u = jax.random.uniform(k1, (_B, a_r, _H, m_r),
                               minval=-log_mult, maxval=log_mult)
        flip = jax.random.bernoulli(k2, 0.5, (1,))
        us.append(jnp.transpose(u, (0, 2, 1, 3)).reshape(_B * _H, a_r, m_r))
        flips.append(flip.astype(jnp.float32).reshape(1, 1, 1))
        bucket *= 2
        scur //= 2
    return us, flips


def kernel(query, key, value):
    B, T, H, E = query.shape
    assert (B, T, H, E) == (_B, _S, _H, _E)
    softmax_temp = 1.0 / math.sqrt(E)
    st = math.sqrt(softmax_temp)

    us, flips = _setup_randomness()

    k_scaled_sq = jnp.sum((key * st) * (key * st), axis=3, keepdims=True)
    shift = jnp.max(k_scaled_sq, axis=1, keepdims=True)
    shift = shift[:, 0, :, 0].reshape(B * H, 1, 1)
    bsqd = (jnp.max(jnp.abs(value), axis=(1, 3), keepdims=True)) ** 2
    bsqd = bsqd[:, 0, :, 0].reshape(B * H, 1, 1)

    bh = B * H
    k_cur = jnp.transpose(key, (0, 2, 1, 3)).reshape(bh, T, E)
    v_cur = jnp.transpose(value, (0, 2, 1, 3)).reshape(bh, T, E)
    for r in range(_ROUNDS):
        s_r = _S >> r
        b = 16 << r
        m = b // 2
        a_r = s_r // b
        in_spec = pl.BlockSpec((1, s_r, E), lambda i: (i, 0, 0))
        in_specs = [
                in_spec,
                in_spec,
                pl.BlockSpec((1, a_r, m), lambda i: (i, 0, 0)),
                pl.BlockSpec((1, 1, 1), lambda i: (i, 0, 0)),
                pl.BlockSpec((1, 1, 1), lambda i: (i, 0, 0)),
                pl.BlockSpec((1, 1, 1), lambda i: (0, 0, 0)),
        ]
        out_specs = [
            pl.BlockSpec((1, s_r // 2, E), lambda i: (i, 0, 0)),
            pl.BlockSpec((1, s_r // 2, E), lambda i: (i, 0, 0)),
        ]
        k_cur, v_cur = pl.pallas_call(
            partial(_round_body, r=r, scale=st if r == 0 else None),
            grid=(bh,),
            in_specs=in_specs,
            out_specs=out_specs,
            out_shape=[
                jax.ShapeDtypeStruct((bh, s_r // 2, E), jnp.float32),
                jax.ShapeDtypeStruct((bh, s_r // 2, E), jnp.float32),
            ],
            scratch_shapes=[
                pltpu.VMEM((a_r, m, m), jnp.float32),
                pltpu.VMEM((a_r, m), jnp.float32),
                pltpu.VMEM((2, 256, 128) if b > 128 else (1, 128, 128),
                           jnp.float32),
                pltpu.VMEM((256, 128) if b > 128 else (128, 128), jnp.float32),
            ],
            compiler_params=pltpu.CompilerParams(
                dimension_semantics=("parallel",),
            ),
        )(k_cur, v_cur, us[r], shift, bsqd, flips[r])

    tq = 1024
    out = pl.pallas_call(
        partial(_attn_body, st=st),
        grid=(B, T // tq),
        in_specs=[
            pl.BlockSpec((1, tq, H, E), lambda b_, t: (b_, t, 0, 0)),
            pl.BlockSpec((H, 128, E), lambda b_, t: (b_, 0, 0)),
            pl.BlockSpec((H, 128, E), lambda b_, t: (b_, 0, 0)),
        ],
        out_specs=pl.BlockSpec((1, tq, H, E), lambda b_, t: (b_, t, 0, 0)),
        out_shape=jax.ShapeDtypeStruct((B, T, H, E), jnp.float32),
        compiler_params=pltpu.CompilerParams(
            dimension_semantics=("parallel", "parallel"),
        ),
    )(query, k_cur, v_cur)
    return out


# speedup vs baseline: 5.2184x; 1.3073x over previous
import math
from functools import partial

import jax
import jax.numpy as jnp
from jax.experimental import pallas as pl
from jax.experimental.pallas import tpu as pltpu

_B, _S, _H, _E = 2, 4096, 16, 64
_ROUNDS = 5
_NEG_INF = float("-inf")


def _compress_common(kget, vget, u, shift, bsqd, kd_put, km_scr, kd1_scr, r):
    b = 16 << r
    m = b // 2
    S = _S >> r
    A = S // b

    def gram(x):
        return jax.lax.dot_general(x, x, (((1,), (1,)), ((), ())))

    def km_of(gk, gv):
        return (gv + bsqd) * jnp.exp(gk - shift)

    def double_diff(km):
        n = km.shape[0]
        nb = n // 128
        for cb in range(nb):
            km_scr[cb] = km[:, cb * 128:(cb + 1) * 128]
        parts = [km_scr[cb, pl.Slice(0, n // 2, 2), :]
                 - km_scr[cb, pl.Slice(1, n // 2, 2), :] for cb in range(nb)]
        p = parts[0] if nb == 1 else jnp.concatenate(parts, axis=1)
        kd1t = jnp.transpose(p, (1, 0))
        if n // 2 == 128:
            kd1_scr[...] = kd1t
        else:
            kd1_scr[:, 0:n // 2] = kd1t
        full = (kd1_scr[pl.Slice(0, n // 2, 2), :]
                - kd1_scr[pl.Slice(1, n // 2, 2), :])
        return full if n // 2 == 128 else full[:, 0:n // 2]

    csd0_parts = []
    diag_parts = []
    if b <= 128:
        g = 128 // b
        eye = (jax.lax.broadcasted_iota(jnp.int32, (64, 64), 0)
               == jax.lax.broadcasted_iota(jnp.int32, (64, 64), 1)
               ).astype(jnp.float32)
        for c in range(S // 128):
            km = km_of(gram(kget(pl.ds(c * 128, 128))),
                       gram(vget(pl.ds(c * 128, 128))))
            kd = double_diff(km)
            dg = jnp.sum(kd * eye, axis=0, keepdims=True)
            blocks = []
            for j in range(g):
                lo = j * m
                blocks.append(kd[lo:lo + m, lo:lo + m].reshape(1, m, m))
                csd0_parts.append(kd[lo:lo + 1, lo:lo + m])
                diag_parts.append(dg[:, lo:lo + m])
            kd_put(pl.ds(c * g, g), jnp.concatenate(blocks, axis=0))
    else:
        km = km_of(gram(kget(pl.ds(0, 256))), gram(vget(pl.ds(0, 256))))
        kd = double_diff(km)
        eye = (jax.lax.broadcasted_iota(jnp.int32, (m, m), 0)
               == jax.lax.broadcasted_iota(jnp.int32, (m, m), 1)
               ).astype(jnp.float32)
        kd_put(pl.ds(0, 1), kd.reshape(1, m, m))
        csd0_parts.append(kd[0:1, :])
        diag_parts.append(jnp.sum(kd * eye, axis=0, keepdims=True))

    csd0 = jnp.concatenate(csd0_parts, axis=0) if A > 1 else csd0_parts[0]
    diag = jnp.concatenate(diag_parts, axis=0) if A > 1 else diag_parts[0]
    rt = jnp.sqrt(diag)

    cmax = rt
    sh = 1
    while sh < m:
        shifted = jnp.concatenate(
            [jnp.full((A, sh), _NEG_INF, jnp.float32), cmax[:, :m - sh]],
            axis=1)
        cmax = jnp.maximum(cmax, shifted)
        sh *= 2
    u2 = u * rt * cmax
    return csd0, u2


def _scan_swaps(csd0, u2, kd_read, flipf, N, m):
    lane = jax.lax.broadcasted_iota(jnp.int32, (N, m), 1)

    def body(i, carry):
        csd, swp = carry
        msk = (lane == i).astype(jnp.float32)
        csd_col = jnp.sum(csd * msk, axis=1, keepdims=True)
        u_col = jnp.sum(u2 * msk, axis=1, keepdims=True)
        spf = (u_col <= csd_col).astype(jnp.float32)
        swp = swp + jnp.abs(spf - flipf) * msk
        return csd + (1.0 - 2.0 * spf) * kd_read(i), swp

    swp0 = jnp.where(lane == 0, flipf, 0.0)
    _, swp = jax.lax.fori_loop(1, m, body, (csd0, swp0))
    return swp


def _select_rows(src, dst, f_col_fn, A, m, loop_over_buckets):
    if loop_over_buckets:
        for a in range(A):
            fa = f_col_fn(a)
            e = src(pl.Slice(2 * a * m, m, 2))
            o = src(pl.Slice(2 * a * m + 1, m, 2))
            dst(pl.ds(a * m, m), jnp.where(fa, o, e))
    else:
        for j in range(m):
            fj = f_col_fn(j)
            e = src(pl.Slice(2 * j, A, 2 * m))
            o = src(pl.Slice(2 * j + 1, A, 2 * m))
            dst(pl.Slice(j, A, m), jnp.where(fj, o, e))


def _fused_round_body(k_ref, v_ref, u_ref, shift_ref, bsqd_ref, flip_ref,
                      ko_ref, vo_ref, kd_scr, f_scr, km_scr, kd1_scr,
                      *extra_scr, r, scale):
    m = (16 << r) // 2
    A = (_S >> r) // (16 << r)
    shift = shift_ref[0, 0, 0]
    bsqd = bsqd_ref[0, 0, 0]
    flipf = flip_ref[0, 0, 0]

    if r == 0:
        kraw, vraw, sem_k, sem_v = extra_scr
        i = pl.program_id(0)
        b_ = i // _H
        hh = i % _H
        cpk = pltpu.make_async_copy(k_ref.at[b_, :, hh, :], kraw, sem_k)
        cpv = pltpu.make_async_copy(v_ref.at[b_, :, hh, :], vraw, sem_v)
        cpk.start()
        cpv.start()
        cpk.wait()
        cpv.wait()
        kget = lambda idx: kraw[idx, :] * scale
        vget = lambda idx: vraw[idx, :]
    else:
        kget = lambda idx: k_ref[0, idx, :]
        vget = lambda idx: v_ref[0, idx, :]

    def kd_put(sl, val):
        kd_scr[sl] = val

    csd0, u2 = _compress_common(kget, vget, u_ref[0], shift, bsqd,
                                kd_put, km_scr, kd1_scr, r)
    kd_read = lambda i: kd_scr[:, pl.ds(i, 1), :].reshape(A, m)
    f_scr[...] = _scan_swaps(csd0, u2, kd_read, flipf, A, m)

    f_col = lambda j: f_scr[:, j:j + 1] > 0.5
    _select_rows(kget, lambda sl, val: ko_ref.__setitem__((0, sl), val),
                 f_col, A, m, False)
    _select_rows(vget, lambda sl, val: vo_ref.__setitem__((0, sl), val),
                 f_col, A, m, False)


def _kd_body(k_ref, v_ref, u_ref, shift_ref, bsqd_ref,
             kd_ref, c0_ref, u2_ref, km_scr, kd1_scr, *, r):
    shift = shift_ref[0, 0, 0]
    bsqd = bsqd_ref[0, 0, 0]
    kget = lambda idx: k_ref[0, idx, :]
    vget = lambda idx: v_ref[0, idx, :]

    def kd_put(sl, val):
        kd_ref[0, sl] = val

    csd0, u2 = _compress_common(kget, vget, u_ref[0], shift, bsqd,
                                kd_put, km_scr, kd1_scr, r)
    c0_ref[0] = csd0
    u2_ref[0] = u2


def _scan_body(kd_ref, c0_ref, u2_ref, flip_ref, f_ref, *, r):
    m = (16 << r) // 2
    A = (_S >> r) // (16 << r)
    N = _B * _H * A
    flipf = flip_ref[0, 0, 0]
    csd0 = c0_ref[...].reshape(N, m)
    u2 = u2_ref[...].reshape(N, m)
    kd_read = lambda i: kd_ref[:, :, pl.ds(i, 1), :].reshape(N, m)
    f_ref[...] = _scan_swaps(csd0, u2, kd_read, flipf, N, m).reshape(
        _B * _H, A, m)


def _select_body(k_ref, v_ref, f_ref, ko_ref, vo_ref, *, r):
    m = (16 << r) // 2
    A = (_S >> r) // (16 << r)

    def f_col(a):
        return jnp.transpose(f_ref[0, a:a + 1, :], (1, 0)) > 0.5

    _select_rows(lambda idx: k_ref[0, idx, :],
                 lambda sl, val: ko_ref.__setitem__((0, sl), val),
                 f_col, A, m, True)
    _select_rows(lambda idx: v_ref[0, idx, :],
                 lambda sl, val: vo_ref.__setitem__((0, sl), val),
                 f_col, A, m, True)


def _attn_body(q_ref, kc_ref, vc_ref, o_ref, *, st):
    for h in range(_H):
        q = q_ref[0, :, h, :] * st
        kc = kc_ref[0, h]
        vc = vc_ref[0, h]
        s = jax.lax.dot_general(q, kc, (((1,), (1,)), ((), ())))
        mx = jnp.max(s, axis=1, keepdims=True)
        p = jnp.exp(s - mx)
        p = p / jnp.sum(p, axis=1, keepdims=True)
        o_ref[0, :, h, :] = jax.lax.dot_general(p, vc, (((1,), (0,)), ((), ())))


def _setup_randomness():
    rng = jax.random.key(42)
    halve_prob = 0.5 / 4096.0
    us, flips = [], []
    bucket, scur = 16, 4096
    for i in range(_ROUNDS):
        sb = bucket
        delta = halve_prob * sb * sb
        log_mult = 0.5 + math.log(2 * sb / delta)
        a_r, m_r = scur // bucket, bucket // 2
        r_i = jax.random.fold_in(rng, i)
        k1, k2 = jax.random.split(r_i)
        u = jax.random.uniform(k1, (_B, a_r, _H, m_r),
                               minval=-log_mult, maxval=log_mult)
        flip = jax.random.bernoulli(k2, 0.5, (1,))
        us.append(jnp.transpose(u, (0, 2, 1, 3)).reshape(_B * _H, a_r, m_r))
        flips.append(flip.astype(jnp.float32).reshape(1, 1, 1))
        bucket *= 2
        scur //= 2
    return us, flips


def _cparams(n):
    return pltpu.CompilerParams(dimension_semantics=("parallel",) * n)


def kernel(query, key, value):
    B, T, H, E = query.shape
    assert (B, T, H, E) == (_B, _S, _H, _E)
    softmax_temp = 1.0 / math.sqrt(E)
    st = math.sqrt(softmax_temp)

    us, flips = _setup_randomness()

    k_scaled_sq = jnp.sum((key * st) * (key * st), axis=3, keepdims=True)
    shift = jnp.max(k_scaled_sq, axis=1, keepdims=True)
    shift = shift[:, 0, :, 0].reshape(B * H, 1, 1)
    bsqd = (jnp.max(jnp.abs(value), axis=(1, 3), keepdims=True)) ** 2
    bsqd = bsqd[:, 0, :, 0].reshape(B * H, 1, 1)

    bh = B * H
    f32 = jnp.float32
    k_cur, v_cur = None, None
    for r in range(_ROUNDS):
        s_r = _S >> r
        b = 16 << r
        m = b // 2
        a_r = s_r // b
        km_scr = pltpu.VMEM((2, 256, 128) if b > 128 else (1, 128, 128), f32)
        kd1_scr = pltpu.VMEM((256, 128) if b > 128 else (128, 128), f32)
        out_shape = [jax.ShapeDtypeStruct((bh, s_r // 2, E), f32),
                     jax.ShapeDtypeStruct((bh, s_r // 2, E), f32)]
        out_specs = [pl.BlockSpec((1, s_r // 2, E), lambda i: (i, 0, 0)),
                     pl.BlockSpec((1, s_r // 2, E), lambda i: (i, 0, 0))]
        if r <= 1:
            if r == 0:
                kv_specs = [
                    pl.BlockSpec(memory_space=pl.ANY),
                    pl.BlockSpec(memory_space=pl.ANY),
                ]
                grid = (bh,)
                ix = lambda i: (i, 0, 0)
                ix0 = lambda i: (0, 0, 0)
                oix = lambda i: (i, 0, 0)
                extra = [pltpu.VMEM((T, E), f32), pltpu.VMEM((T, E), f32),
                         pltpu.SemaphoreType.DMA, pltpu.SemaphoreType.DMA]
                kv_in = (key, value)
            else:
                kv_specs = [pl.BlockSpec((1, s_r, E), lambda i: (i, 0, 0)),
                            pl.BlockSpec((1, s_r, E), lambda i: (i, 0, 0))]
                grid = (bh,)
                ix = lambda i: (i, 0, 0)
                ix0 = lambda i: (0, 0, 0)
                oix = lambda i: (i, 0, 0)
                extra = []
                kv_in = (k_cur, v_cur)
            out_specs = [pl.BlockSpec((1, s_r // 2, E), oix),
                         pl.BlockSpec((1, s_r // 2, E), oix)]
            k_cur, v_cur = pl.pallas_call(
                partial(_fused_round_body, r=r, scale=st if r == 0 else None),
                grid=grid,
                in_specs=kv_specs + [
                    pl.BlockSpec((1, a_r, m), ix),
                    pl.BlockSpec((1, 1, 1), ix),
                    pl.BlockSpec((1, 1, 1), ix),
                    pl.BlockSpec((1, 1, 1), ix0),
                ],
                out_specs=out_specs,
                out_shape=out_shape,
                scratch_shapes=[pltpu.VMEM((a_r, m, m), f32),
                                pltpu.VMEM((a_r, m), f32),
                                km_scr, kd1_scr] + extra,
                compiler_params=_cparams(len(grid)),
            )(*kv_in, us[r], shift, bsqd, flips[r])
        else:
            kv_spec = pl.BlockSpec((1, s_r, E), lambda i: (i, 0, 0))
            kd, c0, u2 = pl.pallas_call(
                partial(_kd_body, r=r),
                grid=(bh,),
                in_specs=[kv_spec, kv_spec,
                          pl.BlockSpec((1, a_r, m), lambda i: (i, 0, 0)),
                          pl.BlockSpec((1, 1, 1), lambda i: (i, 0, 0)),
                          pl.BlockSpec((1, 1, 1), lambda i: (i, 0, 0))],
                out_specs=[
                    pl.BlockSpec((1, a_r, m, m), lambda i: (i, 0, 0, 0)),
                    pl.BlockSpec((1, a_r, m), lambda i: (i, 0, 0)),
                    pl.BlockSpec((1, a_r, m), lambda i: (i, 0, 0)),
                ],
                out_shape=[jax.ShapeDtypeStruct((bh, a_r, m, m), f32),
                           jax.ShapeDtypeStruct((bh, a_r, m), f32),
                           jax.ShapeDtypeStruct((bh, a_r, m), f32)],
                scratch_shapes=[km_scr, kd1_scr],
                compiler_params=_cparams(1),
            )(k_cur, v_cur, us[r], shift, bsqd)
            fbits = pl.pallas_call(
                partial(_scan_body, r=r),
                grid=(1,),
                in_specs=[
                    pl.BlockSpec((bh, a_r, m, m), lambda i: (0, 0, 0, 0)),
                    pl.BlockSpec((bh, a_r, m), lambda i: (0, 0, 0)),
                    pl.BlockSpec((bh, a_r, m), lambda i: (0, 0, 0)),
                    pl.BlockSpec((1, 1, 1), lambda i: (0, 0, 0)),
                ],
                out_specs=pl.BlockSpec((bh, a_r, m), lambda i: (0, 0, 0)),
                out_shape=jax.ShapeDtypeStruct((bh, a_r, m), f32),
                compiler_params=_cparams(1),
            )(kd, c0, u2, flips[r])
            k_cur, v_cur = pl.pallas_call(
                partial(_select_body, r=r),
                grid=(bh,),
                in_specs=[kv_spec, kv_spec,
                          pl.BlockSpec((1, a_r, m), lambda i: (i, 0, 0))],
                out_specs=out_specs,
                out_shape=out_shape,
                compiler_params=_cparams(1),
            )(k_cur, v_cur, fbits)

    tq = 1024
    kc = k_cur.reshape(B, H, 128, E)
    vc = v_cur.reshape(B, H, 128, E)
    out = pl.pallas_call(
        partial(_attn_body, st=st),
        grid=(B, T // tq),
        in_specs=[
            pl.BlockSpec((1, tq, H, E), lambda b_, t: (b_, t, 0, 0)),
            pl.BlockSpec((1, H, 128, E), lambda b_, t: (b_, 0, 0, 0)),
            pl.BlockSpec((1, H, 128, E), lambda b_, t: (b_, 0, 0, 0)),
        ],
        out_specs=pl.BlockSpec((1, tq, H, E), lambda b_, t: (b_, t, 0, 0)),
        out_shape=jax.ShapeDtypeStruct((B, T, H, E), f32),
        compiler_params=_cparams(2),
    )(query, kc, vc)
    return out


# speedup vs baseline: 5.6880x; 1.0900x over previous
import math
from functools import partial

import jax
import jax.numpy as jnp
from jax.experimental import pallas as pl
from jax.experimental.pallas import tpu as pltpu

_B, _S, _H, _E = 2, 4096, 16, 64
_ROUNDS = 5
_NEG_INF = float("-inf")


def _compress_common(kget, vget, u, shift, bsqd, kd_put, km_scr, kd1_scr, r):
    b = 16 << r
    m = b // 2
    S = _S >> r
    A = S // b

    def gram(x):
        return jax.lax.dot_general(x, x, (((1,), (1,)), ((), ())))

    def km_of(gk, gv):
        return (gv + bsqd) * jnp.exp(gk - shift)

    def double_diff(km):
        n = km.shape[0]
        nb = n // 128
        for cb in range(nb):
            km_scr[cb] = km[:, cb * 128:(cb + 1) * 128]
        parts = [km_scr[cb, pl.Slice(0, n // 2, 2), :]
                 - km_scr[cb, pl.Slice(1, n // 2, 2), :] for cb in range(nb)]
        p = parts[0] if nb == 1 else jnp.concatenate(parts, axis=1)
        kd1t = jnp.transpose(p, (1, 0))
        if n // 2 == 128:
            kd1_scr[...] = kd1t
        else:
            kd1_scr[:, 0:n // 2] = kd1t
        full = (kd1_scr[pl.Slice(0, n // 2, 2), :]
                - kd1_scr[pl.Slice(1, n // 2, 2), :])
        return full if n // 2 == 128 else full[:, 0:n // 2]

    csd0_parts = []
    diag_parts = []
    if b <= 128:
        g = 128 // b
        eye = (jax.lax.broadcasted_iota(jnp.int32, (64, 64), 0)
               == jax.lax.broadcasted_iota(jnp.int32, (64, 64), 1)
               ).astype(jnp.float32)
        for c in range(S // 128):
            km = km_of(gram(kget(pl.ds(c * 128, 128))),
                       gram(vget(pl.ds(c * 128, 128))))
            kd = double_diff(km)
            dg = jnp.sum(kd * eye, axis=0, keepdims=True)
            blocks = []
            for j in range(g):
                lo = j * m
                blocks.append(kd[lo:lo + m, lo:lo + m].reshape(1, m, m))
                csd0_parts.append(kd[lo:lo + 1, lo:lo + m])
                diag_parts.append(dg[:, lo:lo + m])
            kd_put(pl.ds(c * g, g), jnp.concatenate(blocks, axis=0))
    else:
        km = km_of(gram(kget(pl.ds(0, 256))), gram(vget(pl.ds(0, 256))))
        kd = double_diff(km)
        eye = (jax.lax.broadcasted_iota(jnp.int32, (m, m), 0)
               == jax.lax.broadcasted_iota(jnp.int32, (m, m), 1)
               ).astype(jnp.float32)
        kd_put(pl.ds(0, 1), kd.reshape(1, m, m))
        csd0_parts.append(kd[0:1, :])
        diag_parts.append(jnp.sum(kd * eye, axis=0, keepdims=True))

    csd0 = jnp.concatenate(csd0_parts, axis=0) if A > 1 else csd0_parts[0]
    diag = jnp.concatenate(diag_parts, axis=0) if A > 1 else diag_parts[0]
    rt = jnp.sqrt(diag)

    cmax = rt
    sh = 1
    while sh < m:
        shifted = jnp.concatenate(
            [jnp.full((A, sh), _NEG_INF, jnp.float32), cmax[:, :m - sh]],
            axis=1)
        cmax = jnp.maximum(cmax, shifted)
        sh *= 2
    u2 = u * rt * cmax
    return csd0, u2


def _scan_swaps(csd0, u2, kd_read, flipf, N, m):
    lane = jax.lax.broadcasted_iota(jnp.int32, (N, m), 1)

    def body(i, carry):
        csd, swp = carry
        msk = (lane == i).astype(jnp.float32)
        csd_col = jnp.sum(csd * msk, axis=1, keepdims=True)
        u_col = jnp.sum(u2 * msk, axis=1, keepdims=True)
        spf = (u_col <= csd_col).astype(jnp.float32)
        swp = swp + jnp.abs(spf - flipf) * msk
        return csd + (1.0 - 2.0 * spf) * kd_read(i), swp

    swp0 = jnp.where(lane == 0, flipf, 0.0)
    _, swp = jax.lax.fori_loop(1, m, body, (csd0, swp0))
    return swp


def _select_rows(src, dst, f_col_fn, A, m, loop_over_buckets):
    if loop_over_buckets:
        for a in range(A):
            fa = f_col_fn(a)
            e = src(pl.Slice(2 * a * m, m, 2))
            o = src(pl.Slice(2 * a * m + 1, m, 2))
            dst(pl.ds(a * m, m), jnp.where(fa, o, e))
    else:
        for j in range(m):
            fj = f_col_fn(j)
            e = src(pl.Slice(2 * j, A, 2 * m))
            o = src(pl.Slice(2 * j + 1, A, 2 * m))
            dst(pl.Slice(j, A, m), jnp.where(fj, o, e))


def _fused_round_body(k_ref, v_ref, u_ref, shift_ref, bsqd_ref, flip_ref,
                      ko_ref, vo_ref, kd_scr, f_scr, km_scr, kd1_scr,
                      *extra_scr, r, scale):
    m = (16 << r) // 2
    A = (_S >> r) // (16 << r)
    shift = shift_ref[0, 0, 0]
    bsqd = bsqd_ref[0, 0, 0]
    flipf = flip_ref[0, 0, 0]

    if r == 0:
        kraw, vraw, sem_k, sem_v = extra_scr
        i = pl.program_id(0)
        b_ = i // _H
        hh = i % _H
        slot = jax.lax.rem(i, 2)
        nslot = jax.lax.rem(i + 1, 2)

        def copies(j, s):
            bj = jax.lax.div(j, _H)
            hj = jax.lax.rem(j, _H)
            return (pltpu.make_async_copy(k_ref.at[bj, :, hj, :],
                                          kraw.at[s], sem_k.at[s]),
                    pltpu.make_async_copy(v_ref.at[bj, :, hj, :],
                                          vraw.at[s], sem_v.at[s]))

        @pl.when(i == 0)
        def _():
            for cp in copies(i, slot):
                cp.start()

        @pl.when(i + 1 < _B * _H)
        def _():
            for cp in copies(i + 1, nslot):
                cp.start()

        for cp in copies(i, slot):
            cp.wait()
        kget = lambda idx: kraw[slot, idx, :] * scale
        vget = lambda idx: vraw[slot, idx, :]
    else:
        kget = lambda idx: k_ref[0, idx, :]
        vget = lambda idx: v_ref[0, idx, :]

    def kd_put(sl, val):
        kd_scr[sl] = val

    csd0, u2 = _compress_common(kget, vget, u_ref[0], shift, bsqd,
                                kd_put, km_scr, kd1_scr, r)
    kd_read = lambda i: kd_scr[:, pl.ds(i, 1), :].reshape(A, m)
    f_scr[...] = _scan_swaps(csd0, u2, kd_read, flipf, A, m)

    f_col = lambda j: f_scr[:, j:j + 1] > 0.5
    _select_rows(kget, lambda sl, val: ko_ref.__setitem__((0, sl), val),
                 f_col, A, m, False)
    _select_rows(vget, lambda sl, val: vo_ref.__setitem__((0, sl), val),
                 f_col, A, m, False)


def _kd_body(k_ref, v_ref, u_ref, shift_ref, bsqd_ref,
             kd_ref, c0_ref, u2_ref, km_scr, kd1_scr, *, r):
    shift = shift_ref[0, 0, 0]
    bsqd = bsqd_ref[0, 0, 0]
    kget = lambda idx: k_ref[0, idx, :]
    vget = lambda idx: v_ref[0, idx, :]

    def kd_put(sl, val):
        kd_ref[0, sl] = val

    csd0, u2 = _compress_common(kget, vget, u_ref[0], shift, bsqd,
                                kd_put, km_scr, kd1_scr, r)
    c0_ref[0] = csd0
    u2_ref[0] = u2


def _scan_body(kd_ref, c0_ref, u2_ref, flip_ref, f_ref, *, r):
    m = (16 << r) // 2
    A = (_S >> r) // (16 << r)
    N = _B * _H * A
    flipf = flip_ref[0, 0, 0]
    csd0 = c0_ref[...].reshape(N, m)
    u2 = u2_ref[...].reshape(N, m)
    kd_read = lambda i: kd_ref[:, :, pl.ds(i, 1), :].reshape(N, m)
    f_ref[...] = _scan_swaps(csd0, u2, kd_read, flipf, N, m).reshape(
        _B * _H, A, m)


def _select_body(k_ref, v_ref, f_ref, ko_ref, vo_ref, *, r):
    m = (16 << r) // 2
    A = (_S >> r) // (16 << r)

    def f_col(a):
        return jnp.transpose(f_ref[0, a:a + 1, :], (1, 0)) > 0.5

    _select_rows(lambda idx: k_ref[0, idx, :],
                 lambda sl, val: ko_ref.__setitem__((0, sl), val),
                 f_col, A, m, True)
    _select_rows(lambda idx: v_ref[0, idx, :],
                 lambda sl, val: vo_ref.__setitem__((0, sl), val),
                 f_col, A, m, True)


def _attn_body(q_ref, kc_ref, vc_ref, o_ref, *, st):
    for h in range(_H):
        q = q_ref[0, :, h, :] * st
        kc = kc_ref[0, h]
        vc = vc_ref[0, h]
        s = jax.lax.dot_general(q, kc, (((1,), (1,)), ((), ())))
        mx = jnp.max(s, axis=1, keepdims=True)
        p = jnp.exp(s - mx)
        p = p / jnp.sum(p, axis=1, keepdims=True)
        o_ref[0, :, h, :] = jax.lax.dot_general(p, vc, (((1,), (0,)), ((), ())))


def _setup_randomness():
    rng = jax.random.key(42)
    halve_prob = 0.5 / 4096.0
    us, flips = [], []
    bucket, scur = 16, 4096
    for i in range(_ROUNDS):
        sb = bucket
        delta = halve_prob * sb * sb
        log_mult = 0.5 + math.log(2 * sb / delta)
        a_r, m_r = scur // bucket, bucket // 2
        r_i = jax.random.fold_in(rng, i)
        k1, k2 = jax.random.split(r_i)
        u = jax.random.uniform(k1, (_B, a_r, _H, m_r),
                               minval=-log_mult, maxval=log_mult)
        flip = jax.random.bernoulli(k2, 0.5, (1,))
        us.append(jnp.transpose(u, (0, 2, 1, 3)).reshape(_B * _H, a_r, m_r))
        flips.append(flip.astype(jnp.float32).reshape(1, 1, 1))
        bucket *= 2
        scur //= 2
    return us, flips


def _cparams(n):
    return pltpu.CompilerParams(dimension_semantics=("parallel",) * n)


def kernel(query, key, value):
    B, T, H, E = query.shape
    assert (B, T, H, E) == (_B, _S, _H, _E)
    softmax_temp = 1.0 / math.sqrt(E)
    st = math.sqrt(softmax_temp)

    us, flips = _setup_randomness()

    k_scaled_sq = jnp.sum((key * st) * (key * st), axis=3, keepdims=True)
    shift = jnp.max(k_scaled_sq, axis=1, keepdims=True)
    shift = shift[:, 0, :, 0].reshape(B * H, 1, 1)
    bsqd = (jnp.max(jnp.abs(value), axis=(1, 3), keepdims=True)) ** 2
    bsqd = bsqd[:, 0, :, 0].reshape(B * H, 1, 1)

    bh = B * H
    f32 = jnp.float32
    k_cur, v_cur = None, None
    for r in range(_ROUNDS):
        s_r = _S >> r
        b = 16 << r
        m = b // 2
        a_r = s_r // b
        km_scr = pltpu.VMEM((2, 256, 128) if b > 128 else (1, 128, 128), f32)
        kd1_scr = pltpu.VMEM((256, 128) if b > 128 else (128, 128), f32)
        out_shape = [jax.ShapeDtypeStruct((bh, s_r // 2, E), f32),
                     jax.ShapeDtypeStruct((bh, s_r // 2, E), f32)]
        out_specs = [pl.BlockSpec((1, s_r // 2, E), lambda i: (i, 0, 0)),
                     pl.BlockSpec((1, s_r // 2, E), lambda i: (i, 0, 0))]
        if r <= 1:
            if r == 0:
                kv_specs = [
                    pl.BlockSpec(memory_space=pl.ANY),
                    pl.BlockSpec(memory_space=pl.ANY),
                ]
                grid = (bh,)
                ix = lambda i: (i, 0, 0)
                ix0 = lambda i: (0, 0, 0)
                oix = lambda i: (i, 0, 0)
                extra = [pltpu.VMEM((2, T, E), f32), pltpu.VMEM((2, T, E), f32),
                         pltpu.SemaphoreType.DMA((2,)),
                         pltpu.SemaphoreType.DMA((2,))]
                kv_in = (key, value)
            else:
                kv_specs = [pl.BlockSpec((1, s_r, E), lambda i: (i, 0, 0)),
                            pl.BlockSpec((1, s_r, E), lambda i: (i, 0, 0))]
                grid = (bh,)
                ix = lambda i: (i, 0, 0)
                ix0 = lambda i: (0, 0, 0)
                oix = lambda i: (i, 0, 0)
                extra = []
                kv_in = (k_cur, v_cur)
            out_specs = [pl.BlockSpec((1, s_r // 2, E), oix),
                         pl.BlockSpec((1, s_r // 2, E), oix)]
            k_cur, v_cur = pl.pallas_call(
                partial(_fused_round_body, r=r, scale=st if r == 0 else None),
                grid=grid,
                in_specs=kv_specs + [
                    pl.BlockSpec((1, a_r, m), ix),
                    pl.BlockSpec((1, 1, 1), ix),
                    pl.BlockSpec((1, 1, 1), ix),
                    pl.BlockSpec((1, 1, 1), ix0),
                ],
                out_specs=out_specs,
                out_shape=out_shape,
                scratch_shapes=[pltpu.VMEM((a_r, m, m), f32),
                                pltpu.VMEM((a_r, m), f32),
                                km_scr, kd1_scr] + extra,
                compiler_params=_cparams(len(grid)),
            )(*kv_in, us[r], shift, bsqd, flips[r])
        else:
            kv_spec = pl.BlockSpec((1, s_r, E), lambda i: (i, 0, 0))
            kd, c0, u2 = pl.pallas_call(
                partial(_kd_body, r=r),
                grid=(bh,),
                in_specs=[kv_spec, kv_spec,
                          pl.BlockSpec((1, a_r, m), lambda i: (i, 0, 0)),
                          pl.BlockSpec((1, 1, 1), lambda i: (i, 0, 0)),
                          pl.BlockSpec((1, 1, 1), lambda i: (i, 0, 0))],
                out_specs=[
                    pl.BlockSpec((1, a_r, m, m), lambda i: (i, 0, 0, 0)),
                    pl.BlockSpec((1, a_r, m), lambda i: (i, 0, 0)),
                    pl.BlockSpec((1, a_r, m), lambda i: (i, 0, 0)),
                ],
                out_shape=[jax.ShapeDtypeStruct((bh, a_r, m, m), f32),
                           jax.ShapeDtypeStruct((bh, a_r, m), f32),
                           jax.ShapeDtypeStruct((bh, a_r, m), f32)],
                scratch_shapes=[km_scr, kd1_scr],
                compiler_params=_cparams(1),
            )(k_cur, v_cur, us[r], shift, bsqd)
            fbits = pl.pallas_call(
                partial(_scan_body, r=r),
                grid=(1,),
                in_specs=[
                    pl.BlockSpec((bh, a_r, m, m), lambda i: (0, 0, 0, 0)),
                    pl.BlockSpec((bh, a_r, m), lambda i: (0, 0, 0)),
                    pl.BlockSpec((bh, a_r, m), lambda i: (0, 0, 0)),
                    pl.BlockSpec((1, 1, 1), lambda i: (0, 0, 0)),
                ],
                out_specs=pl.BlockSpec((bh, a_r, m), lambda i: (0, 0, 0)),
                out_shape=jax.ShapeDtypeStruct((bh, a_r, m), f32),
                compiler_params=_cparams(1),
            )(kd, c0, u2, flips[r])
            k_cur, v_cur = pl.pallas_call(
                partial(_select_body, r=r),
                grid=(bh,),
                in_specs=[kv_spec, kv_spec,
                          pl.BlockSpec((1, a_r, m), lambda i: (i, 0, 0))],
                out_specs=out_specs,
                out_shape=out_shape,
                compiler_params=_cparams(1),
            )(k_cur, v_cur, fbits)

    tq = 1024
    kc = k_cur.reshape(B, H, 128, E)
    vc = v_cur.reshape(B, H, 128, E)
    out = pl.pallas_call(
        partial(_attn_body, st=st),
        grid=(B, T // tq),
        in_specs=[
            pl.BlockSpec((1, tq, H, E), lambda b_, t: (b_, t, 0, 0)),
            pl.BlockSpec((1, H, 128, E), lambda b_, t: (b_, 0, 0, 0)),
            pl.BlockSpec((1, H, 128, E), lambda b_, t: (b_, 0, 0, 0)),
        ],
        out_specs=pl.BlockSpec((1, tq, H, E), lambda b_, t: (b_, t, 0, 0)),
        out_shape=jax.ShapeDtypeStruct((B, T, H, E), f32),
        compiler_params=_cparams(2),
    )(query, kc, vc)
    return out


# speedup vs baseline: 5.7043x; 1.0029x over previous
import math
from functools import partial

import jax
import jax.numpy as jnp
from jax.experimental import pallas as pl
from jax.experimental.pallas import tpu as pltpu

_B, _S, _H, _E = 2, 4096, 16, 64
_ROUNDS = 5
_NEG_INF = float("-inf")


def _compress_common(kget, vget, u, shift, bsqd, kd_put, km_scr, kd1_scr, r):
    b = 16 << r
    m = b // 2
    S = _S >> r
    A = S // b

    def gram(x):
        return jax.lax.dot_general(x, x, (((1,), (1,)), ((), ())))

    def km_of(gk, gv):
        return (gv + bsqd) * jnp.exp(gk - shift)

    def double_diff(km):
        n = km.shape[0]
        nb = n // 128
        for cb in range(nb):
            km_scr[cb] = km[:, cb * 128:(cb + 1) * 128]
        parts = [km_scr[cb, pl.Slice(0, n // 2, 2), :]
                 - km_scr[cb, pl.Slice(1, n // 2, 2), :] for cb in range(nb)]
        p = parts[0] if nb == 1 else jnp.concatenate(parts, axis=1)
        kd1t = jnp.transpose(p, (1, 0))
        if n // 2 == 128:
            kd1_scr[...] = kd1t
        else:
            kd1_scr[:, 0:n // 2] = kd1t
        full = (kd1_scr[pl.Slice(0, n // 2, 2), :]
                - kd1_scr[pl.Slice(1, n // 2, 2), :])
        return full if n // 2 == 128 else full[:, 0:n // 2]

    csd0_parts = []
    diag_parts = []
    if b <= 128:
        g = 128 // b
        eye = (jax.lax.broadcasted_iota(jnp.int32, (64, 64), 0)
               == jax.lax.broadcasted_iota(jnp.int32, (64, 64), 1)
               ).astype(jnp.float32)
        for c in range(S // 128):
            km = km_of(gram(kget(pl.ds(c * 128, 128))),
                       gram(vget(pl.ds(c * 128, 128))))
            kd = double_diff(km)
            dg = jnp.sum(kd * eye, axis=0, keepdims=True)
            blocks = []
            for j in range(g):
                lo = j * m
                blocks.append(kd[lo:lo + m, lo:lo + m].reshape(1, m, m))
                csd0_parts.append(kd[lo:lo + 1, lo:lo + m])
                diag_parts.append(dg[:, lo:lo + m])
            kd_put(pl.ds(c * g, g), jnp.concatenate(blocks, axis=0))
    else:
        km = km_of(gram(kget(pl.ds(0, 256))), gram(vget(pl.ds(0, 256))))
        kd = double_diff(km)
        eye = (jax.lax.broadcasted_iota(jnp.int32, (m, m), 0)
               == jax.lax.broadcasted_iota(jnp.int32, (m, m), 1)
               ).astype(jnp.float32)
        kd_put(pl.ds(0, 1), kd.reshape(1, m, m))
        csd0_parts.append(kd[0:1, :])
        diag_parts.append(jnp.sum(kd * eye, axis=0, keepdims=True))

    csd0 = jnp.concatenate(csd0_parts, axis=0) if A > 1 else csd0_parts[0]
    diag = jnp.concatenate(diag_parts, axis=0) if A > 1 else diag_parts[0]
    rt = jnp.sqrt(diag)

    cmax = rt
    sh = 1
    while sh < m:
        shifted = jnp.concatenate(
            [jnp.full((A, sh), _NEG_INF, jnp.float32), cmax[:, :m - sh]],
            axis=1)
        cmax = jnp.maximum(cmax, shifted)
        sh *= 2
    u2 = u * rt * cmax
    return csd0, u2


def _scan_swaps(csd0, u2, kd_read, flipf, N, m):
    lane = jax.lax.broadcasted_iota(jnp.int32, (N, m), 1)

    def body(i, carry):
        csd, swp = carry
        msk = (lane == i).astype(jnp.float32)
        csd_col = jnp.sum(csd * msk, axis=1, keepdims=True)
        u_col = jnp.sum(u2 * msk, axis=1, keepdims=True)
        spf = (u_col <= csd_col).astype(jnp.float32)
        swp = swp + jnp.abs(spf - flipf) * msk
        return csd + (1.0 - 2.0 * spf) * kd_read(i), swp

    swp0 = jnp.where(lane == 0, flipf, 0.0)
    _, swp = jax.lax.fori_loop(1, m, body, (csd0, swp0))
    return swp


def _select_rows(src, dst, f_col_fn, A, m, loop_over_buckets):
    if loop_over_buckets:
        for a in range(A):
            fa = f_col_fn(a)
            e = src(pl.Slice(2 * a * m, m, 2))
            o = src(pl.Slice(2 * a * m + 1, m, 2))
            dst(pl.ds(a * m, m), jnp.where(fa, o, e))
    else:
        for j in range(m):
            fj = f_col_fn(j)
            e = src(pl.Slice(2 * j, A, 2 * m))
            o = src(pl.Slice(2 * j + 1, A, 2 * m))
            dst(pl.Slice(j, A, m), jnp.where(fj, o, e))


def _fused_round_body(k_ref, v_ref, u_ref, shift_ref, bsqd_ref, flip_ref,
                      ko_ref, vo_ref, kd_scr, f_scr, km_scr, kd1_scr,
                      *extra_scr, r, scale):
    m = (16 << r) // 2
    A = (_S >> r) // (16 << r)
    shift = shift_ref[0, 0, 0]
    bsqd = bsqd_ref[0, 0, 0]
    flipf = flip_ref[0, 0, 0]

    if r == 0:
        kraw, vraw, sem_k, sem_v = extra_scr
        i = pl.program_id(0)
        b_ = i // _H
        hh = i % _H
        slot = jax.lax.rem(i, 2)
        nslot = jax.lax.rem(i + 1, 2)

        def copies(j, s):
            bj = jax.lax.div(j, _H)
            hj = jax.lax.rem(j, _H)
            return (pltpu.make_async_copy(k_ref.at[bj, :, hj, :],
                                          kraw.at[s], sem_k.at[s]),
                    pltpu.make_async_copy(v_ref.at[bj, :, hj, :],
                                          vraw.at[s], sem_v.at[s]))

        @pl.when(i == 0)
        def _():
            for cp in copies(i, slot):
                cp.start()

        @pl.when(i + 1 < _B * _H)
        def _():
            for cp in copies(i + 1, nslot):
                cp.start()

        for cp in copies(i, slot):
            cp.wait()
        kget = lambda idx: kraw[slot, idx, :] * scale
        vget = lambda idx: vraw[slot, idx, :]
    else:
        kget = lambda idx: k_ref[0, idx, :]
        vget = lambda idx: v_ref[0, idx, :]

    def kd_put(sl, val):
        kd_scr[sl] = val

    csd0, u2 = _compress_common(kget, vget, u_ref[0], shift, bsqd,
                                kd_put, km_scr, kd1_scr, r)
    kd_read = lambda i: kd_scr[:, pl.ds(i, 1), :].reshape(A, m)
    f_scr[...] = _scan_swaps(csd0, u2, kd_read, flipf, A, m)

    f_col = lambda j: f_scr[:, j:j + 1] > 0.5
    _select_rows(kget, lambda sl, val: ko_ref.__setitem__((0, sl), val),
                 f_col, A, m, False)
    _select_rows(vget, lambda sl, val: vo_ref.__setitem__((0, sl), val),
                 f_col, A, m, False)


def _kd_body(k_ref, v_ref, u_ref, shift_ref, bsqd_ref,
             kd_ref, c0_ref, u2_ref, km_scr, kd1_scr, *, r):
    shift = shift_ref[0, 0, 0]
    bsqd = bsqd_ref[0, 0, 0]
    kget = lambda idx: k_ref[0, idx, :]
    vget = lambda idx: v_ref[0, idx, :]

    def kd_put(sl, val):
        kd_ref[0, sl] = val

    csd0, u2 = _compress_common(kget, vget, u_ref[0], shift, bsqd,
                                kd_put, km_scr, kd1_scr, r)
    c0_ref[0] = csd0
    u2_ref[0] = u2


def _scan_body(kd_ref, c0_ref, u2_ref, flip_ref, f_ref, *, r):
    m = (16 << r) // 2
    A = (_S >> r) // (16 << r)
    N = _B * _H * A
    flipf = flip_ref[0, 0, 0]
    csd0 = c0_ref[...].reshape(N, m)
    u2 = u2_ref[...].reshape(N, m)
    kd_read = lambda i: kd_ref[:, :, pl.ds(i, 1), :].reshape(N, m)
    f_ref[...] = _scan_swaps(csd0, u2, kd_read, flipf, N, m).reshape(
        _B * _H, A, m)


def _select_body(k_ref, v_ref, f_ref, ko_ref, vo_ref, *, r):
    m = (16 << r) // 2
    A = (_S >> r) // (16 << r)
    by_bucket = A <= m

    if by_bucket:
        def f_col(a):
            return jnp.transpose(f_ref[0, a:a + 1, :], (1, 0)) > 0.5
    else:
        def f_col(j):
            return f_ref[0, :, j:j + 1] > 0.5

    _select_rows(lambda idx: k_ref[0, idx, :],
                 lambda sl, val: ko_ref.__setitem__((0, sl), val),
                 f_col, A, m, by_bucket)
    _select_rows(lambda idx: v_ref[0, idx, :],
                 lambda sl, val: vo_ref.__setitem__((0, sl), val),
                 f_col, A, m, by_bucket)


def _attn_body(q_ref, kc_ref, vc_ref, o_ref, *, st):
    for h in range(_H):
        q = q_ref[0, :, h, :] * st
        kc = kc_ref[0, h]
        vc = vc_ref[0, h]
        s = jax.lax.dot_general(q, kc, (((1,), (1,)), ((), ())))
        mx = jnp.max(s, axis=1, keepdims=True)
        p = jnp.exp(s - mx)
        p = p / jnp.sum(p, axis=1, keepdims=True)
        o_ref[0, :, h, :] = jax.lax.dot_general(p, vc, (((1,), (0,)), ((), ())))


def _setup_randomness():
    rng = jax.random.key(42)
    halve_prob = 0.5 / 4096.0
    us, flips = [], []
    bucket, scur = 16, 4096
    for i in range(_ROUNDS):
        sb = bucket
        delta = halve_prob * sb * sb
        log_mult = 0.5 + math.log(2 * sb / delta)
        a_r, m_r = scur // bucket, bucket // 2
        r_i = jax.random.fold_in(rng, i)
        k1, k2 = jax.random.split(r_i)
        u = jax.random.uniform(k1, (_B, a_r, _H, m_r),
                               minval=-log_mult, maxval=log_mult)
        flip = jax.random.bernoulli(k2, 0.5, (1,))
        us.append(jnp.transpose(u, (0, 2, 1, 3)).reshape(_B * _H, a_r, m_r))
        flips.append(flip.astype(jnp.float32).reshape(1, 1, 1))
        bucket *= 2
        scur //= 2
    return us, flips


def _cparams(n):
    return pltpu.CompilerParams(dimension_semantics=("parallel",) * n)


def kernel(query, key, value):
    B, T, H, E = query.shape
    assert (B, T, H, E) == (_B, _S, _H, _E)
    softmax_temp = 1.0 / math.sqrt(E)
    st = math.sqrt(softmax_temp)

    us, flips = _setup_randomness()

    k_scaled_sq = jnp.sum((key * st) * (key * st), axis=3, keepdims=True)
    shift = jnp.max(k_scaled_sq, axis=1, keepdims=True)
    shift = shift[:, 0, :, 0].reshape(B * H, 1, 1)
    bsqd = (jnp.max(jnp.abs(value), axis=(1, 3), keepdims=True)) ** 2
    bsqd = bsqd[:, 0, :, 0].reshape(B * H, 1, 1)

    bh = B * H
    f32 = jnp.float32
    k_cur, v_cur = None, None
    for r in range(_ROUNDS):
        s_r = _S >> r
        b = 16 << r
        m = b // 2
        a_r = s_r // b
        km_scr = pltpu.VMEM((2, 256, 128) if b > 128 else (1, 128, 128), f32)
        kd1_scr = pltpu.VMEM((256, 128) if b > 128 else (128, 128), f32)
        out_shape = [jax.ShapeDtypeStruct((bh, s_r // 2, E), f32),
                     jax.ShapeDtypeStruct((bh, s_r // 2, E), f32)]
        out_specs = [pl.BlockSpec((1, s_r // 2, E), lambda i: (i, 0, 0)),
                     pl.BlockSpec((1, s_r // 2, E), lambda i: (i, 0, 0))]
        if r == 0:
            if r == 0:
                kv_specs = [
                    pl.BlockSpec(memory_space=pl.ANY),
                    pl.BlockSpec(memory_space=pl.ANY),
                ]
                grid = (bh,)
                ix = lambda i: (i, 0, 0)
                ix0 = lambda i: (0, 0, 0)
                oix = lambda i: (i, 0, 0)
                extra = [pltpu.VMEM((2, T, E), f32), pltpu.VMEM((2, T, E), f32),
                         pltpu.SemaphoreType.DMA((2,)),
                         pltpu.SemaphoreType.DMA((2,))]
                kv_in = (key, value)
            else:
                kv_specs = [pl.BlockSpec((1, s_r, E), lambda i: (i, 0, 0)),
                            pl.BlockSpec((1, s_r, E), lambda i: (i, 0, 0))]
                grid = (bh,)
                ix = lambda i: (i, 0, 0)
                ix0 = lambda i: (0, 0, 0)
                oix = lambda i: (i, 0, 0)
                extra = []
                kv_in = (k_cur, v_cur)
            out_specs = [pl.BlockSpec((1, s_r // 2, E), oix),
                         pl.BlockSpec((1, s_r // 2, E), oix)]
            k_cur, v_cur = pl.pallas_call(
                partial(_fused_round_body, r=r, scale=st if r == 0 else None),
                grid=grid,
                in_specs=kv_specs + [
                    pl.BlockSpec((1, a_r, m), ix),
                    pl.BlockSpec((1, 1, 1), ix),
                    pl.BlockSpec((1, 1, 1), ix),
                    pl.BlockSpec((1, 1, 1), ix0),
                ],
                out_specs=out_specs,
                out_shape=out_shape,
                scratch_shapes=[pltpu.VMEM((a_r, m, m), f32),
                                pltpu.VMEM((a_r, m), f32),
                                km_scr, kd1_scr] + extra,
                compiler_params=_cparams(len(grid)),
            )(*kv_in, us[r], shift, bsqd, flips[r])
        else:
            kv_spec = pl.BlockSpec((1, s_r, E), lambda i: (i, 0, 0))
            kd, c0, u2 = pl.pallas_call(
                partial(_kd_body, r=r),
                grid=(bh,),
                in_specs=[kv_spec, kv_spec,
                          pl.BlockSpec((1, a_r, m), lambda i: (i, 0, 0)),
                          pl.BlockSpec((1, 1, 1), lambda i: (i, 0, 0)),
                          pl.BlockSpec((1, 1, 1), lambda i: (i, 0, 0))],
                out_specs=[
                    pl.BlockSpec((1, a_r, m, m), lambda i: (i, 0, 0, 0)),
                    pl.BlockSpec((1, a_r, m), lambda i: (i, 0, 0)),
                    pl.BlockSpec((1, a_r, m), lambda i: (i, 0, 0)),
                ],
                out_shape=[jax.ShapeDtypeStruct((bh, a_r, m, m), f32),
                           jax.ShapeDtypeStruct((bh, a_r, m), f32),
                           jax.ShapeDtypeStruct((bh, a_r, m), f32)],
                scratch_shapes=[km_scr, kd1_scr],
                compiler_params=_cparams(1),
            )(k_cur, v_cur, us[r], shift, bsqd)
            fbits = pl.pallas_call(
                partial(_scan_body, r=r),
                grid=(1,),
                in_specs=[
                    pl.BlockSpec((bh, a_r, m, m), lambda i: (0, 0, 0, 0)),
                    pl.BlockSpec((bh, a_r, m), lambda i: (0, 0, 0)),
                    pl.BlockSpec((bh, a_r, m), lambda i: (0, 0, 0)),
                    pl.BlockSpec((1, 1, 1), lambda i: (0, 0, 0)),
                ],
                out_specs=pl.BlockSpec((bh, a_r, m), lambda i: (0, 0, 0)),
                out_shape=jax.ShapeDtypeStruct((bh, a_r, m), f32),
                compiler_params=_cparams(1),
            )(kd, c0, u2, flips[r])
            k_cur, v_cur = pl.pallas_call(
                partial(_select_body, r=r),
                grid=(bh,),
                in_specs=[kv_spec, kv_spec,
                          pl.BlockSpec((1, a_r, m), lambda i: (i, 0, 0))],
                out_specs=out_specs,
                out_shape=out_shape,
                compiler_params=_cparams(1),
            )(k_cur, v_cur, fbits)

    tq = 1024
    kc = k_cur.reshape(B, H, 128, E)
    vc = v_cur.reshape(B, H, 128, E)
    out = pl.pallas_call(
        partial(_attn_body, st=st),
        grid=(B, T // tq),
        in_specs=[
            pl.BlockSpec((1, tq, H, E), lambda b_, t: (b_, t, 0, 0)),
            pl.BlockSpec((1, H, 128, E), lambda b_, t: (b_, 0, 0, 0)),
            pl.BlockSpec((1, H, 128, E), lambda b_, t: (b_, 0, 0, 0)),
        ],
        out_specs=pl.BlockSpec((1, tq, H, E), lambda b_, t: (b_, t, 0, 0)),
        out_shape=jax.ShapeDtypeStruct((B, T, H, E), f32),
        compiler_params=_cparams(2),
    )(query, kc, vc)
    return out
